# Initial kernel scaffold; baseline (speedup 1.0000x reference)
import jax
import jax.numpy as jnp
from jax.experimental import pallas as pl
from jax.experimental.pallas import tpu as pltpu


def _posemb_kernel(wr_ref, out_ref, t_ref, sems):
    g = pl.program_id(0)
    ng = pl.num_programs(0)

    @pl.when(g == 0)
    def _build_tables():
        f32 = jnp.float32
        w = wr_ref[:]
        ri = jax.lax.broadcasted_iota(jnp.int32, (128, 128), 0)
        ci = jax.lax.broadcasted_iota(jnp.int32, (128, 128), 1)

        def dot(a, b):
            return jax.lax.dot(a, b, preferred_element_type=f32)

        perm = (ri == (8 * (15 - ci // 8) + ci % 8)).astype(f32)
        flip = ((ri + ci) == 127).astype(f32)
        rev = dot(flip, dot(w, perm))
        roll8 = (ri == ((ci + 8) % 128)).astype(f32)
        gfw = dot(w, roll8)
        pick = ((ri == 127) & (ci == 0)).astype(f32)
        b0 = dot(pick, gfw)
        low = jnp.where((ri < 127) | (ci < 120), rev, b0)
        gup = jnp.concatenate([gfw[1:], jnp.zeros((1, 128), f32)], axis=0)
        high = jnp.where(ci < 120, gfw, gup)
        wext = jnp.concatenate([low, high], axis=0)

        lane256 = jax.lax.broadcasted_iota(jnp.int32, (256, 128), 1)
        for t in range(16):
            sh = (15 - t) * 8
            rollm = (ri == ((ci + sh) % 128)).astype(f32)
            rolled = dot(wext, rollm)
            rollup = jnp.concatenate(
                [rolled[1:], jnp.zeros((1, 128), f32)], axis=0)
            t_ref[t] = jnp.where(lane256 < (128 - sh), rolled, rollup)

    def copy_for(step):
        src = t_ref.at[:, pl.ds(127 - step, 128), :]
        dst = out_ref.at[pl.ds(step * 16, 16)]
        return pltpu.make_async_copy(src, dst, sems.at[step % 2])

    cur = copy_for(g)
    cur.start()

    @pl.when(g > 0)
    def _wait_prev():
        copy_for(g - 1).wait()

    @pl.when(g == ng - 1)
    def _wait_last():
        cur.wait()


def kernel(x, W):
    bs, _, seq_len = x.shape
    num, out = W.shape
    assert seq_len == 2048 and num == 2048 and out == 8
    wr = W.reshape(128, 128)
    res = pl.pallas_call(
        _posemb_kernel,
        grid=(128,),
        in_specs=[pl.BlockSpec((128, 128), lambda g: (0, 0))],
        out_specs=pl.BlockSpec(memory_space=pltpu.MemorySpace.ANY),
        out_shape=jax.ShapeDtypeStruct((2048, 128, 128), jnp.float32),
        scratch_shapes=[
            pltpu.VMEM((16, 256, 128), jnp.float32),
            pltpu.SemaphoreType.DMA((2,)),
        ],
    )(wr)
    emb = res.reshape(1, out, seq_len, seq_len)
    if bs > 1:
        emb = jnp.tile(emb, (bs, 1, 1, 1))
    return emb


# speedup vs baseline: 101.7386x; 101.7386x over previous
import jax
import jax.numpy as jnp
from jax.experimental import pallas as pl
from jax.experimental.pallas import tpu as pltpu


def _posemb_kernel(wr_ref, out_ref, t_ref, sems):
    g = pl.program_id(0)
    ng = pl.num_programs(0)

    @pl.when(g == 0)
    def _build_tables():
        f32 = jnp.float32
        w = wr_ref[:]
        ri = jax.lax.broadcasted_iota(jnp.int32, (128, 128), 0)
        ci = jax.lax.broadcasted_iota(jnp.int32, (128, 128), 1)

        def dot(a, b):
            return jax.lax.dot(a, b, preferred_element_type=f32)

        perm = (ri == (8 * (15 - ci // 8) + ci % 8)).astype(f32)
        flip = ((ri + ci) == 127).astype(f32)
        rev = dot(flip, dot(w, perm))
        roll8 = (ri == ((ci + 8) % 128)).astype(f32)
        gfw = dot(w, roll8)
        pick = ((ri == 127) & (ci == 0)).astype(f32)
        b0 = dot(pick, gfw)
        low = jnp.where((ri < 127) | (ci < 120), rev, b0)
        gup = jnp.concatenate([gfw[1:], jnp.zeros((1, 128), f32)], axis=0)
        high = jnp.where(ci < 120, gfw, gup)
        wext = jnp.concatenate([low, high], axis=0)

        lane256 = jax.lax.broadcasted_iota(jnp.int32, (256, 128), 1)
        for t in range(16):
            sh = (15 - t) * 8
            rollm = (ri == ((ci + sh) % 128)).astype(f32)
            rolled = dot(wext, rollm)
            rollup = jnp.concatenate(
                [rolled[1:], jnp.zeros((1, 128), f32)], axis=0)
            t_ref[t] = jnp.where(lane256 < (128 - sh), rolled, rollup)

    def copy_for(step):
        src = t_ref.at[:, pl.ds(127 - step, 128), :]
        dst = out_ref.at[pl.ds(step * 16, 16)]
        return pltpu.make_async_copy(src, dst, sems.at[step % 2])

    cur = copy_for(g)
    cur.start()

    @pl.when(g > 0)
    def _wait_prev():
        copy_for(g - 1).wait()

    @pl.when(g == ng - 1)
    def _wait_last():
        cur.wait()


def kernel(x, W):
    bs, _, seq_len = x.shape
    num, out = W.shape
    assert seq_len == 2048 and num == 2048 and out == 8
    wr = W.reshape(128, 128)
    res = pl.pallas_call(
        _posemb_kernel,
        grid=(128,),
        in_specs=[pl.BlockSpec((128, 128), lambda g: (0, 0))],
        out_specs=pl.BlockSpec(memory_space=pl.ANY),
        out_shape=jax.ShapeDtypeStruct((2048, 128, 128), jnp.float32),
        scratch_shapes=[
            pltpu.VMEM((16, 256, 128), jnp.float32),
            pltpu.SemaphoreType.DMA((2,)),
        ],
    )(wr)
    emb = res.reshape(1, out, seq_len, seq_len)
    if bs > 1:
        emb = jnp.tile(emb, (bs, 1, 1, 1))
    return emb


# speedup vs baseline: 114.0134x; 1.1207x over previous
import jax
import jax.numpy as jnp
from jax.experimental import pallas as pl
from jax.experimental.pallas import tpu as pltpu


_NBUF = 8


def _posemb_kernel(wr_ref, out_ref, t_ref, sems):
    g = pl.program_id(0)
    ng = pl.num_programs(0)

    @pl.when(g == 0)
    def _build_tables():
        f32 = jnp.float32
        w = wr_ref[:]
        ri = jax.lax.broadcasted_iota(jnp.int32, (128, 128), 0)
        ci = jax.lax.broadcasted_iota(jnp.int32, (128, 128), 1)

        def dot(a, b):
            return jax.lax.dot(a, b, preferred_element_type=f32,
                               precision=jax.lax.Precision.HIGHEST)

        perm = (ri == (8 * (15 - ci // 8) + ci % 8)).astype(f32)
        flip = ((ri + ci) == 127).astype(f32)
        rev = dot(flip, dot(w, perm))
        roll8 = (ri == ((ci + 8) % 128)).astype(f32)
        gfw = dot(w, roll8)
        pick = ((ri == 127) & (ci == 0)).astype(f32)
        b0 = dot(pick, gfw)
        low = jnp.where((ri < 127) | (ci < 120), rev, b0)
        gup = jnp.concatenate([gfw[1:], jnp.zeros((1, 128), f32)], axis=0)
        high = jnp.where(ci < 120, gfw, gup)
        wext = jnp.concatenate([low, high], axis=0)

        lane256 = jax.lax.broadcasted_iota(jnp.int32, (256, 128), 1)
        for t in range(16):
            sh = (15 - t) * 8
            rollm = (ri == ((ci + sh) % 128)).astype(f32)
            rolled = dot(wext, rollm)
            rollup = jnp.concatenate(
                [rolled[1:], jnp.zeros((1, 128), f32)], axis=0)
            t_ref[t] = jnp.where(lane256 < (128 - sh), rolled, rollup)

    def copy_for(step):
        src = t_ref.at[:, pl.ds(127 - step, 128), :]
        dst = out_ref.at[pl.ds(step * 16, 16)]
        return pltpu.make_async_copy(src, dst, sems.at[step % _NBUF])

    @pl.when(g >= _NBUF)
    def _wait_oldest():
        copy_for(g - _NBUF).wait()

    copy_for(g).start()

    @pl.when(g == ng - 1)
    def _drain():
        for k in range(_NBUF):
            copy_for(ng - _NBUF + k).wait()


def kernel(x, W):
    bs, _, seq_len = x.shape
    num, out = W.shape
    assert seq_len == 2048 and num == 2048 and out == 8
    wr = W.reshape(128, 128)
    res = pl.pallas_call(
        _posemb_kernel,
        grid=(128,),
        in_specs=[pl.BlockSpec((128, 128), lambda g: (0, 0))],
        out_specs=pl.BlockSpec(memory_space=pl.ANY),
        out_shape=jax.ShapeDtypeStruct((2048, 128, 128), jnp.float32),
        scratch_shapes=[
            pltpu.VMEM((16, 256, 128), jnp.float32),
            pltpu.SemaphoreType.DMA((_NBUF,)),
        ],
    )(wr)
    emb = res.reshape(1, out, seq_len, seq_len)
    if bs > 1:
        emb = jnp.tile(emb, (bs, 1, 1, 1))
    return emb


# speedup vs baseline: 114.3955x; 1.0034x over previous
import jax
import jax.numpy as jnp
from jax.experimental import pallas as pl
from jax.experimental.pallas import tpu as pltpu


_NQ = 4
_NBUF = 2


def _posemb_kernel(wr_ref, out_ref, t_ref, sems):
    g = pl.program_id(0)
    ng = pl.num_programs(0)

    @pl.when(g == 0)
    def _build_tables():
        f32 = jnp.float32
        w = wr_ref[:]
        ri = jax.lax.broadcasted_iota(jnp.int32, (128, 128), 0)
        ci = jax.lax.broadcasted_iota(jnp.int32, (128, 128), 1)

        def dot(a, b):
            return jax.lax.dot(a, b, preferred_element_type=f32,
                               precision=jax.lax.Precision.HIGHEST)

        perm = (ri == (8 * (15 - ci // 8) + ci % 8)).astype(f32)
        flip = ((ri + ci) == 127).astype(f32)
        rev = dot(flip, dot(w, perm))
        roll8 = (ri == ((ci + 8) % 128)).astype(f32)
        gfw = dot(w, roll8)
        pick = ((ri == 127) & (ci == 0)).astype(f32)
        b0 = dot(pick, gfw)
        low = jnp.where((ri < 127) | (ci < 120), rev, b0)
        gup = jnp.concatenate([gfw[1:], jnp.zeros((1, 128), f32)], axis=0)
        high = jnp.where(ci < 120, gfw, gup)
        wext = jnp.concatenate([low, high], axis=0)

        lane256 = jax.lax.broadcasted_iota(jnp.int32, (256, 128), 1)
        for t in range(16):
            sh = (15 - t) * 8
            rollm = (ri == ((ci + sh) % 128)).astype(f32)
            rolled = dot(wext, rollm)
            rollup = jnp.concatenate(
                [rolled[1:], jnp.zeros((1, 128), f32)], axis=0)
            t_ref[t] = jnp.where(lane256 < (128 - sh), rolled, rollup)

    def copy_for(q, m):
        step = m * _NQ + q
        src = t_ref.at[:, pl.ds(127 - step, 128), :]
        dst = out_ref.at[pl.ds(step * 16, 16)]
        return pltpu.make_async_copy(src, dst, sems.at[q, m % _NBUF])

    for q in range(_NQ):
        @pl.when(g >= _NBUF)
        def _wait_oldest(q=q):
            copy_for(q, g - _NBUF).wait()

        copy_for(q, g).start()

    @pl.when(g == ng - 1)
    def _drain():
        for q in range(_NQ):
            for k in range(_NBUF):
                copy_for(q, ng - _NBUF + k).wait()


def kernel(x, W):
    bs, _, seq_len = x.shape
    num, out = W.shape
    assert seq_len == 2048 and num == 2048 and out == 8
    wr = W.reshape(128, 128)
    res = pl.pallas_call(
        _posemb_kernel,
        grid=(128 // _NQ,),
        in_specs=[pl.BlockSpec((128, 128), lambda g: (0, 0))],
        out_specs=pl.BlockSpec(memory_space=pl.ANY),
        out_shape=jax.ShapeDtypeStruct((2048, 128, 128), jnp.float32),
        scratch_shapes=[
            pltpu.VMEM((16, 256, 128), jnp.float32),
            pltpu.SemaphoreType.DMA((_NQ, _NBUF)),
        ],
    )(wr)
    emb = res.reshape(1, out, seq_len, seq_len)
    if bs > 1:
        emb = jnp.tile(emb, (bs, 1, 1, 1))
    return emb
